# Initial kernel scaffold; baseline (speedup 1.0000x reference)
import jax
import jax.numpy as jnp
from jax.experimental import pallas as pl


def kernel(nodes, neighbors, aspects, W, Wa, ba, s_src, s_tgt, bias):
    raise NotImplementedError("write your pallas kernel here")


# speedup vs baseline: 2.7538x; 2.7538x over previous
import functools

import jax
import jax.numpy as jnp
from jax.experimental import pallas as pl

N, K, D = 1024, 32, 128
BLK = 128


def _body(nodes_ref, neigh_ref, asp_ref, W_ref, Wa_ref, ba_ref, ssrc_ref,
          stgt_ref, bias_ref, out_ref):
    W = W_ref[...]
    Wa = Wa_ref[...]
    svec = ssrc_ref[...]
    stgt = stgt_ref[...]
    ba = ba_ref[...]

    hi = jax.lax.Precision.HIGHEST
    v = jnp.dot(svec, Wa, precision=hi, preferred_element_type=jnp.float32)
    w1 = jnp.dot(v[:, :D], W, precision=hi, preferred_element_type=jnp.float32)
    w2 = jnp.dot(v[:, D:], W, precision=hi, preferred_element_type=jnp.float32)
    wt = jnp.dot(stgt, W, precision=hi, preferred_element_type=jnp.float32)
    c = jnp.sum(svec * ba)

    neigh = neigh_ref[...]
    asp = asp_ref[...]
    nodes = nodes_ref[...]

    s1 = jnp.sum(neigh * w1[0][None, None, :], axis=-1)
    s2 = jnp.sum(asp * w2[0][None, None, :], axis=-1)
    st = jnp.sum(nodes * wt, axis=-1)

    scores = s1 + s2 + c + st[:, None]
    scores = jnp.where(scores >= 0, scores, 0.2 * scores)
    e = jnp.exp(scores)
    attn = e / (jnp.sum(e, axis=1, keepdims=True) + 1e-16)

    agg = jnp.sum(neigh * attn[..., None], axis=1)
    out = jax.lax.dot_general(agg, W, (((1,), (1,)), ((), ())),
                              precision=jax.lax.Precision.HIGHEST,
                              preferred_element_type=jnp.float32)
    out = out + bias_ref[...]
    out_ref[...] = jnp.where(out > 0, out, jnp.exp(jnp.minimum(out, 0.0)) - 1.0)


@jax.jit
def kernel(nodes, neighbors, aspects, W, Wa, ba, s_src, s_tgt, bias):
    grid = (N // BLK,)
    return pl.pallas_call(
        _body,
        grid=grid,
        in_specs=[
            pl.BlockSpec((BLK, D), lambda i: (i, 0)),
            pl.BlockSpec((BLK, K, D), lambda i: (i, 0, 0)),
            pl.BlockSpec((BLK, K, D), lambda i: (i, 0, 0)),
            pl.BlockSpec((D, D), lambda i: (0, 0)),
            pl.BlockSpec((D, 2 * D), lambda i: (0, 0)),
            pl.BlockSpec((1, D), lambda i: (0, 0)),
            pl.BlockSpec((1, D), lambda i: (0, 0)),
            pl.BlockSpec((1, D), lambda i: (0, 0)),
            pl.BlockSpec((1, D), lambda i: (0, 0)),
        ],
        out_specs=pl.BlockSpec((BLK, D), lambda i: (i, 0)),
        out_shape=jax.ShapeDtypeStruct((N, D), jnp.float32),
    )(nodes, neighbors, aspects, W, Wa, ba.reshape(1, D),
      s_src.reshape(1, D), s_tgt.reshape(1, D), bias.reshape(1, D))
